# baseline (speedup 1.0000x reference)
import jax
import jax.numpy as jnp
from jax import lax
from jax.experimental import pallas as pl
from jax.experimental.pallas import tpu as pltpu
from jax.experimental.pallas import tpu_sc as plsc

B, C, N, K = 16, 3, 4096, 64
R = B * N
NW = 32
ROWS_PER_W = R // NW
G = 128
NG0, NG1 = 21, 11
F = 30


def _rsqrt(v):
    i = plsc.bitcast(v, jnp.int32)
    i = jnp.full((16,), 0x5F3759DF, jnp.int32) - lax.shift_right_arithmetic(
        i, jnp.full((16,), 1, jnp.int32))
    y = plsc.bitcast(i, jnp.float32)
    half = jnp.full((16,), 0.5, jnp.float32)
    three_half = jnp.full((16,), 1.5, jnp.float32)
    for _ in range(3):
        y = y * (three_half - half * v * y * y)
    return y


def _sc_body(gx_hbm, nx_hbm, out_hbm, xyzb, cb, ob, accx, accy, accz,
             accn, sem, osem):
    cid = lax.axis_index("c")
    sid = lax.axis_index("s")
    ng = jnp.where(cid == 0, NG0, NG1)
    start = jnp.where(cid == 0, sid * NG0, NG0 * 16 + sid * NG1)
    iota = lax.iota(jnp.int32, 16)
    ones = jnp.full((16,), 1.0, jnp.float32)
    zeros = jnp.zeros((16,), jnp.float32)
    abase = iota + jnp.full((16,), 112, jnp.int32)
    c25 = jnp.full((16,), 25, jnp.int32)
    c26 = jnp.full((16,), 26, jnp.int32)
    c27 = jnp.full((16,), 27, jnp.int32)
    m64 = jnp.full((16,), 64, jnp.int32)
    m32 = jnp.full((16,), 32, jnp.int32)
    m16 = jnp.full((16,), 16, jnp.int32)
    m63 = jnp.full((16,), 63, jnp.int32)
    i1 = jnp.full((16,), 1, jnp.int32)

    def group_body(g, _):
        gidx = start + g
        b = lax.shift_right_logical(gidx, 5)
        nrow = (gidx & (N // G - 1)) * G
        h0 = pltpu.async_copy(gx_hbm.at[b, :, pl.ds(nrow, G), :], xyzb, sem)
        h1 = pltpu.async_copy(nx_hbm.at[b, pl.ds(nrow, G), :], cb, sem)
        h0.wait()
        h1.wait()
        @pl.when(g > 0)
        def _():
            pltpu.make_async_copy(
                ob, out_hbm.at[b, pl.ds(nrow, G), :], osem).wait()

        def sub_body(s, _):
            r = s * 16 + iota
            rb = r * jnp.full((16,), 128, jnp.int32)
            zi = jnp.zeros((16,), jnp.int32)
            poy = jnp.full((16,), G * 128, jnp.int32)
            poz = jnp.full((16,), 2 * G * 128, jnp.int32)

            for ref in (accx, accy, accz, accn):
                for i in range(8):
                    ref[pl.ds(i * 16, 16)] = zeros

            def point_step(kv, sxx, syy, szz):
                idx = rb + kv
                x = plsc.load_gather(xyzb, [zi, zi, idx])
                y = plsc.load_gather(xyzb, [zi, zi, idx + poy])
                z = plsc.load_gather(xyzb, [zi, zi, idx + poz])
                sx = lax.shift_right_logical(
                    plsc.bitcast(x, jnp.int32), c25) & m64
                sy = lax.shift_right_logical(
                    plsc.bitcast(y, jnp.int32), c26) & m32
                sz = lax.shift_right_logical(
                    plsc.bitcast(z, jnp.int32), c27) & m16
                a = abase - (sx + sy + sz)
                plsc.addupdate_scatter(accx, [a], x)
                plsc.addupdate_scatter(accy, [a], y)
                plsc.addupdate_scatter(accz, [a], z)
                plsc.addupdate_scatter(accn, [a], ones)
                return ((kv + i1) & m63, sxx + x * x, syy + y * y,
                        szz + z * z)

            UNROLL = 8
            def k_body(_, carry):
                kv, sxx, syy, szz = carry
                for _ in range(UNROLL):
                    kv, sxx, syy, szz = point_step(kv, sxx, syy, szz)
                return kv, sxx, syy, szz

            _, sxx, syy, szz = lax.fori_loop(
                0, K // UNROLL, k_body, (iota, zeros, zeros, zeros))

            tot = [zeros, zeros, zeros]
            for o in range(8):
                cnt = accn[pl.ds(o * 16, 16)]
                inv = ones / jnp.maximum(cnt, ones)
                for c, ref in enumerate((accx, accy, accz)):
                    v = ref[pl.ds(o * 16, 16)]
                    tot[c] = tot[c] + v
                    col = rb + jnp.full((16,), 6 + o * 3 + c, jnp.int32)
                    plsc.store_scatter(ob, [zi, col], v * inv)
            sx, sy, sz = tot

            inv_k = jnp.full((16,), 1.0 / K, jnp.float32)
            inv_km1 = jnp.full((16,), 1.0 / (K - 1), jnp.float32)
            tiny = jnp.full((16,), 1e-30, jnp.float32)
            for c, (s1, s2) in enumerate(((sx, sxx), (sy, syy), (sz, szz))):
                var = (s2 - s1 * (s1 * inv_k)) * inv_km1
                var = jnp.maximum(var, zeros)
                std = var * _rsqrt(jnp.maximum(var, tiny))
                plsc.store_scatter(
                    ob, [zi, rb + jnp.full((16,), c, jnp.int32)], std)

            for c in range(3):
                v = plsc.load_gather(
                    cb, [zi, rb + jnp.full((16,), c, jnp.int32)])
                plsc.store_scatter(
                    ob, [zi, rb + jnp.full((16,), 3 + c, jnp.int32)], v)
            return 0

        lax.fori_loop(0, G // 16, sub_body, 0)
        pltpu.async_copy(ob, out_hbm.at[b, pl.ds(nrow, G), :], osem)
        return 0

    lax.fori_loop(0, ng, group_body, 0)
    pltpu.make_async_copy(
        ob, out_hbm.at[0, pl.ds(0, G), :], osem).wait()


@jax.jit
def kernel(group_xyz, new_xyz):
    mesh = plsc.VectorSubcoreMesh(core_axis_name="c", subcore_axis_name="s")
    run = pl.kernel(
        _sc_body,
        out_type=jax.ShapeDtypeStruct((B, N, F), jnp.float32),
        mesh=mesh,
        compiler_params=pltpu.CompilerParams(needs_layout_passes=False),
        scratch_types=[
            pltpu.VMEM((C, G, K), jnp.float32),
            pltpu.VMEM((G, 3), jnp.float32),
            pltpu.VMEM((G, F), jnp.float32),
            pltpu.VMEM((128,), jnp.float32),
            pltpu.VMEM((128,), jnp.float32),
            pltpu.VMEM((128,), jnp.float32),
            pltpu.VMEM((128,), jnp.float32),
            pltpu.SemaphoreType.DMA,
            pltpu.SemaphoreType.DMA,
        ],
    )
    return run(group_xyz, new_xyz)


# speedup vs baseline: 1.0009x; 1.0009x over previous
import jax
import jax.numpy as jnp
from jax import lax
from jax.experimental import pallas as pl
from jax.experimental.pallas import tpu as pltpu
from jax.experimental.pallas import tpu_sc as plsc

B, C, N, K = 16, 3, 4096, 64
R = B * N
NW = 32
ROWS_PER_W = R // NW
G = 128
NG0, NG1 = 11, 21
F = 30


def _rsqrt(v):
    i = plsc.bitcast(v, jnp.int32)
    i = jnp.full((16,), 0x5F3759DF, jnp.int32) - lax.shift_right_arithmetic(
        i, jnp.full((16,), 1, jnp.int32))
    y = plsc.bitcast(i, jnp.float32)
    half = jnp.full((16,), 0.5, jnp.float32)
    three_half = jnp.full((16,), 1.5, jnp.float32)
    for _ in range(3):
        y = y * (three_half - half * v * y * y)
    return y


def _sc_body(gx_hbm, nx_hbm, out_hbm, xyzb, cb, ob, accx, accy, accz,
             accn, sem, osem):
    cid = lax.axis_index("c")
    sid = lax.axis_index("s")
    ng = jnp.where(cid == 0, NG0, NG1)
    start = jnp.where(cid == 0, sid * NG0, NG0 * 16 + sid * NG1)
    iota = lax.iota(jnp.int32, 16)
    ones = jnp.full((16,), 1.0, jnp.float32)
    zeros = jnp.zeros((16,), jnp.float32)
    abase = iota + jnp.full((16,), 112, jnp.int32)
    c25 = jnp.full((16,), 25, jnp.int32)
    c26 = jnp.full((16,), 26, jnp.int32)
    c27 = jnp.full((16,), 27, jnp.int32)
    m64 = jnp.full((16,), 64, jnp.int32)
    m32 = jnp.full((16,), 32, jnp.int32)
    m16 = jnp.full((16,), 16, jnp.int32)
    m63 = jnp.full((16,), 63, jnp.int32)
    i1 = jnp.full((16,), 1, jnp.int32)

    def group_body(g, _):
        gidx = start + g
        b = lax.shift_right_logical(gidx, 5)
        nrow = (gidx & (N // G - 1)) * G
        h0 = pltpu.async_copy(gx_hbm.at[b, :, pl.ds(nrow, G), :], xyzb, sem)
        h1 = pltpu.async_copy(nx_hbm.at[b, pl.ds(nrow, G), :], cb, sem)
        h0.wait()
        h1.wait()
        @pl.when(g > 0)
        def _():
            pltpu.make_async_copy(
                ob, out_hbm.at[b, pl.ds(nrow, G), :], osem).wait()

        def sub_body(s, _):
            r = s * 16 + iota
            rb = r * jnp.full((16,), 128, jnp.int32)
            zi = jnp.zeros((16,), jnp.int32)
            poy = jnp.full((16,), G * 128, jnp.int32)
            poz = jnp.full((16,), 2 * G * 128, jnp.int32)

            for ref in (accx, accy, accz, accn):
                for i in range(8):
                    ref[pl.ds(i * 16, 16)] = zeros

            def point_step(kv, sxx, syy, szz):
                idx = rb + kv
                x = plsc.load_gather(xyzb, [zi, zi, idx])
                y = plsc.load_gather(xyzb, [zi, zi, idx + poy])
                z = plsc.load_gather(xyzb, [zi, zi, idx + poz])
                sx = lax.shift_right_logical(
                    plsc.bitcast(x, jnp.int32), c25) & m64
                sy = lax.shift_right_logical(
                    plsc.bitcast(y, jnp.int32), c26) & m32
                sz = lax.shift_right_logical(
                    plsc.bitcast(z, jnp.int32), c27) & m16
                a = abase - (sx + sy + sz)
                plsc.addupdate_scatter(accx, [a], x)
                plsc.addupdate_scatter(accy, [a], y)
                plsc.addupdate_scatter(accz, [a], z)
                plsc.addupdate_scatter(accn, [a], ones)
                return ((kv + i1) & m63, sxx + x * x, syy + y * y,
                        szz + z * z)

            UNROLL = 8
            def k_body(_, carry):
                kv, sxx, syy, szz = carry
                for _ in range(UNROLL):
                    kv, sxx, syy, szz = point_step(kv, sxx, syy, szz)
                return kv, sxx, syy, szz

            _, sxx, syy, szz = lax.fori_loop(
                0, K // UNROLL, k_body, (iota, zeros, zeros, zeros))

            tot = [zeros, zeros, zeros]
            for o in range(8):
                cnt = accn[pl.ds(o * 16, 16)]
                inv = ones / jnp.maximum(cnt, ones)
                for c, ref in enumerate((accx, accy, accz)):
                    v = ref[pl.ds(o * 16, 16)]
                    tot[c] = tot[c] + v
                    col = rb + jnp.full((16,), 6 + o * 3 + c, jnp.int32)
                    plsc.store_scatter(ob, [zi, col], v * inv)
            sx, sy, sz = tot

            inv_k = jnp.full((16,), 1.0 / K, jnp.float32)
            inv_km1 = jnp.full((16,), 1.0 / (K - 1), jnp.float32)
            tiny = jnp.full((16,), 1e-30, jnp.float32)
            for c, (s1, s2) in enumerate(((sx, sxx), (sy, syy), (sz, szz))):
                var = (s2 - s1 * (s1 * inv_k)) * inv_km1
                var = jnp.maximum(var, zeros)
                std = var * _rsqrt(jnp.maximum(var, tiny))
                plsc.store_scatter(
                    ob, [zi, rb + jnp.full((16,), c, jnp.int32)], std)

            for c in range(3):
                v = plsc.load_gather(
                    cb, [zi, rb + jnp.full((16,), c, jnp.int32)])
                plsc.store_scatter(
                    ob, [zi, rb + jnp.full((16,), 3 + c, jnp.int32)], v)
            return 0

        lax.fori_loop(0, G // 16, sub_body, 0)
        pltpu.async_copy(ob, out_hbm.at[b, pl.ds(nrow, G), :], osem)
        return 0

    lax.fori_loop(0, ng, group_body, 0)
    pltpu.make_async_copy(
        ob, out_hbm.at[0, pl.ds(0, G), :], osem).wait()


@jax.jit
def kernel(group_xyz, new_xyz):
    mesh = plsc.VectorSubcoreMesh(core_axis_name="c", subcore_axis_name="s")
    run = pl.kernel(
        _sc_body,
        out_type=jax.ShapeDtypeStruct((B, N, F), jnp.float32),
        mesh=mesh,
        compiler_params=pltpu.CompilerParams(needs_layout_passes=False),
        scratch_types=[
            pltpu.VMEM((C, G, K), jnp.float32),
            pltpu.VMEM((G, 3), jnp.float32),
            pltpu.VMEM((G, F), jnp.float32),
            pltpu.VMEM((128,), jnp.float32),
            pltpu.VMEM((128,), jnp.float32),
            pltpu.VMEM((128,), jnp.float32),
            pltpu.VMEM((128,), jnp.float32),
            pltpu.SemaphoreType.DMA,
            pltpu.SemaphoreType.DMA,
        ],
    )
    return run(group_xyz, new_xyz)


# speedup vs baseline: 1.1735x; 1.1725x over previous
import jax
import jax.numpy as jnp
from jax import lax
from jax.experimental import pallas as pl
from jax.experimental.pallas import tpu as pltpu
from jax.experimental.pallas import tpu_sc as plsc

B, C, N, K = 16, 3, 4096, 64
R = B * N
NW = 32
ROWS_PER_W = R // NW
G = 128
NGROUPS = ROWS_PER_W // G
F = 30


def _rsqrt(v):
    i = plsc.bitcast(v, jnp.int32)
    i = jnp.full((16,), 0x5F3759DF, jnp.int32) - lax.shift_right_arithmetic(
        i, jnp.full((16,), 1, jnp.int32))
    y = plsc.bitcast(i, jnp.float32)
    half = jnp.full((16,), 0.5, jnp.float32)
    three_half = jnp.full((16,), 1.5, jnp.float32)
    for _ in range(3):
        y = y * (three_half - half * v * y * y)
    return y


def _sc_body(gx_hbm, nx_hbm, out_hbm, xyzb, cb, ob, accx, accy, accz,
             accn, sem, osem):
    cid = lax.axis_index("c")
    sid = lax.axis_index("s")
    wid = sid * 2 + cid
    b = wid // 2
    n0 = (wid % 2) * ROWS_PER_W
    iota = lax.iota(jnp.int32, 16)
    ones = jnp.full((16,), 1.0, jnp.float32)
    zeros = jnp.zeros((16,), jnp.float32)
    abase = iota + jnp.full((16,), 112, jnp.int32)
    c25 = jnp.full((16,), 25, jnp.int32)
    c26 = jnp.full((16,), 26, jnp.int32)
    c27 = jnp.full((16,), 27, jnp.int32)
    m64 = jnp.full((16,), 64, jnp.int32)
    m32 = jnp.full((16,), 32, jnp.int32)
    m16 = jnp.full((16,), 16, jnp.int32)
    m63 = jnp.full((16,), 63, jnp.int32)
    i1 = jnp.full((16,), 1, jnp.int32)

    def group_body(g, _):
        nrow = n0 + g * G
        h0 = pltpu.async_copy(gx_hbm.at[b, :, pl.ds(nrow, G), :], xyzb, sem)
        h1 = pltpu.async_copy(nx_hbm.at[b, pl.ds(nrow, G), :], cb, sem)
        h0.wait()
        h1.wait()
        @pl.when(g > 0)
        def _():
            pltpu.make_async_copy(
                ob, out_hbm.at[b, pl.ds(nrow, G), :], osem).wait()

        def sub_body(s, _):
            r = s * 16 + iota
            rb = r * jnp.full((16,), 128, jnp.int32)
            zi = jnp.zeros((16,), jnp.int32)
            poy = jnp.full((16,), G * 128, jnp.int32)
            poz = jnp.full((16,), 2 * G * 128, jnp.int32)

            for ref in (accx, accy, accz, accn):
                for i in range(8):
                    ref[pl.ds(i * 16, 16)] = zeros

            def point_step(kv, sxx, syy, szz):
                idx = rb + kv
                x = plsc.load_gather(xyzb, [zi, zi, idx])
                y = plsc.load_gather(xyzb, [zi, zi, idx + poy])
                z = plsc.load_gather(xyzb, [zi, zi, idx + poz])
                sx = lax.shift_right_logical(
                    plsc.bitcast(x, jnp.int32), c25) & m64
                sy = lax.shift_right_logical(
                    plsc.bitcast(y, jnp.int32), c26) & m32
                sz = lax.shift_right_logical(
                    plsc.bitcast(z, jnp.int32), c27) & m16
                a = abase - (sx + sy + sz)
                plsc.addupdate_scatter(accx, [a], x)
                plsc.addupdate_scatter(accy, [a], y)
                plsc.addupdate_scatter(accz, [a], z)
                plsc.addupdate_scatter(accn, [a], ones)
                return ((kv + i1) & m63, sxx + x * x, syy + y * y,
                        szz + z * z)

            UNROLL = 8
            def k_body(_, carry):
                kv, sxx, syy, szz = carry
                for _ in range(UNROLL):
                    kv, sxx, syy, szz = point_step(kv, sxx, syy, szz)
                return kv, sxx, syy, szz

            _, sxx, syy, szz = lax.fori_loop(
                0, K // UNROLL, k_body, (iota, zeros, zeros, zeros))

            tot = [zeros, zeros, zeros]
            for o in range(8):
                cnt = accn[pl.ds(o * 16, 16)]
                inv = ones / jnp.maximum(cnt, ones)
                for c, ref in enumerate((accx, accy, accz)):
                    v = ref[pl.ds(o * 16, 16)]
                    tot[c] = tot[c] + v
                    col = rb + jnp.full((16,), 6 + o * 3 + c, jnp.int32)
                    plsc.store_scatter(ob, [zi, col], v * inv)
            sx, sy, sz = tot

            inv_k = jnp.full((16,), 1.0 / K, jnp.float32)
            inv_km1 = jnp.full((16,), 1.0 / (K - 1), jnp.float32)
            tiny = jnp.full((16,), 1e-30, jnp.float32)
            for c, (s1, s2) in enumerate(((sx, sxx), (sy, syy), (sz, szz))):
                var = (s2 - s1 * (s1 * inv_k)) * inv_km1
                var = jnp.maximum(var, zeros)
                std = var * _rsqrt(jnp.maximum(var, tiny))
                plsc.store_scatter(
                    ob, [zi, rb + jnp.full((16,), c, jnp.int32)], std)

            for c in range(3):
                v = plsc.load_gather(
                    cb, [zi, rb + jnp.full((16,), c, jnp.int32)])
                plsc.store_scatter(
                    ob, [zi, rb + jnp.full((16,), 3 + c, jnp.int32)], v)
            return 0

        lax.fori_loop(0, G // 16, sub_body, 0)
        pltpu.async_copy(ob, out_hbm.at[b, pl.ds(nrow, G), :], osem)
        return 0

    lax.fori_loop(0, NGROUPS, group_body, 0)
    pltpu.make_async_copy(
        ob, out_hbm.at[b, pl.ds(n0, G), :], osem).wait()


@jax.jit
def kernel(group_xyz, new_xyz):
    mesh = plsc.VectorSubcoreMesh(core_axis_name="c", subcore_axis_name="s")
    run = pl.kernel(
        _sc_body,
        out_type=jax.ShapeDtypeStruct((B, N, F), jnp.float32),
        mesh=mesh,
        compiler_params=pltpu.CompilerParams(needs_layout_passes=False),
        scratch_types=[
            pltpu.VMEM((C, G, K), jnp.float32),
            pltpu.VMEM((G, 3), jnp.float32),
            pltpu.VMEM((G, F), jnp.float32),
            pltpu.VMEM((128,), jnp.float32),
            pltpu.VMEM((128,), jnp.float32),
            pltpu.VMEM((128,), jnp.float32),
            pltpu.VMEM((128,), jnp.float32),
            pltpu.SemaphoreType.DMA,
            pltpu.SemaphoreType.DMA,
        ],
    )
    return run(group_xyz, new_xyz)
